# initial kernel scaffold (unmeasured)
import jax
import jax.numpy as jnp
from jax import lax
from jax.experimental import pallas as pl
from jax.experimental.pallas import tpu as pltpu

N_DEV = 4
SQ = 2048
SKV_SHARD = 2048
HQ = 8
DH = 128
DM = HQ * DH
BQ = 128
BLK = 64
SCALE = 0.08838834764831843
NEG = -1e9


def _body(x_ref, wq_ref, k_ref, v_ref, wo_ref, out_ref,
          kg_ref, vg_ref, q_ref, ctx_ref,
          copy_sems, ksend, krecv, vsend, vrecv):
    me = lax.axis_index("i")
    left = lax.rem(me + N_DEV - 1, N_DEV)
    right = lax.rem(me + 1, N_DEV)

    barrier_sem = pltpu.get_barrier_semaphore()
    pl.semaphore_signal(barrier_sem, inc=1, device_id=(left,),
                        device_id_type=pl.DeviceIdType.MESH)
    pl.semaphore_signal(barrier_sem, inc=1, device_id=(right,),
                        device_id_type=pl.DeviceIdType.MESH)
    pl.semaphore_wait(barrier_sem, 2)

    k_loc = pltpu.make_async_copy(k_ref, kg_ref.at[me], copy_sems.at[0])
    v_loc = pltpu.make_async_copy(v_ref, vg_ref.at[me], copy_sems.at[1])
    k_loc.start()
    v_loc.start()

    for h in range(N_DEV - 1):
        src_slot = lax.rem(me - h + N_DEV, N_DEV)
        in_slot = lax.rem(me - h - 1 + N_DEV, N_DEV)
        k_src = k_ref if h == 0 else kg_ref.at[src_slot]
        v_src = v_ref if h == 0 else vg_ref.at[src_slot]
        k_send = pltpu.make_async_remote_copy(
            src_ref=k_src, dst_ref=kg_ref.at[src_slot],
            send_sem=ksend.at[h], recv_sem=krecv.at[h],
            device_id=(right,), device_id_type=pl.DeviceIdType.MESH)
        v_send = pltpu.make_async_remote_copy(
            src_ref=v_src, dst_ref=vg_ref.at[src_slot],
            send_sem=vsend.at[h], recv_sem=vrecv.at[h],
            device_id=(right,), device_id_type=pl.DeviceIdType.MESH)
        k_send.start()
        v_send.start()

        if h == 0:
            q = jnp.dot(x_ref[...], wq_ref[...],
                        preferred_element_type=jnp.float32)
            for hh in range(HQ):
                q_ref[hh] = q[:, hh * DH:(hh + 1) * DH].astype(jnp.bfloat16)

        k_recv = pltpu.make_async_remote_copy(
            src_ref=kg_ref.at[in_slot], dst_ref=kg_ref.at[in_slot],
            send_sem=ksend.at[h], recv_sem=krecv.at[h],
            device_id=(left,), device_id_type=pl.DeviceIdType.MESH)
        v_recv = pltpu.make_async_remote_copy(
            src_ref=vg_ref.at[in_slot], dst_ref=vg_ref.at[in_slot],
            send_sem=vsend.at[h], recv_sem=vrecv.at[h],
            device_id=(left,), device_id_type=pl.DeviceIdType.MESH)
        k_recv.wait_recv()
        v_recv.wait_recv()
        k_send.wait_send()
        v_send.wait_send()

    k_loc.wait()
    v_loc.wait()

    n_qb = SQ // BQ

    def qb_step(qb, carry):
        row_blk = (me * SQ + qb * BQ
                   + lax.broadcasted_iota(jnp.int32, (BQ, SKV_SHARD), 0)) // BLK
        col = lax.broadcasted_iota(jnp.int32, (BQ, SKV_SHARD), 1)
        biases = []
        for o in range(N_DEV):
            col_blk = (o * SKV_SHARD + col) // BLK
            keep = ((row_blk == col_blk) | (col_blk == 0)
                    | (lax.rem(row_blk + col_blk, 3) == 0))
            biases.append(jnp.where(keep, 0.0, NEG).astype(jnp.float32))

        def h_step(h, hcarry):
            q = q_ref[h, pl.ds(qb * BQ, BQ), :]
            ss = []
            for o in range(N_DEV):
                s = lax.dot_general(
                    q, kg_ref[o, h], (((1,), (1,)), ((), ())),
                    preferred_element_type=jnp.float32)
                ss.append(s * SCALE + biases[o])
            mx = ss[0].max(axis=1, keepdims=True)
            for s in ss[1:]:
                mx = jnp.maximum(mx, s.max(axis=1, keepdims=True))
            ws = [jnp.exp(s - mx) for s in ss]
            denom = ws[0].sum(axis=1, keepdims=True)
            for w in ws[1:]:
                denom = denom + w.sum(axis=1, keepdims=True)
            acc = jnp.zeros((BQ, DH), jnp.float32)
            for o in range(N_DEV):
                wb = (ws[o] / denom).astype(jnp.bfloat16)
                acc = acc + jnp.dot(wb, vg_ref[o, h],
                                    preferred_element_type=jnp.float32)
            ctx_ref[h, pl.ds(qb * BQ, BQ), :] = acc.astype(jnp.bfloat16)
            return hcarry

        lax.fori_loop(0, HQ, h_step, 0)
        return carry

    lax.fori_loop(0, n_qb, qb_step, 0)

    acc = jnp.zeros((SQ, DM), jnp.float32)
    for h in range(HQ):
        acc = acc + jnp.dot(ctx_ref[h], wo_ref[h * DH:(h + 1) * DH, :],
                            preferred_element_type=jnp.float32)
    out_ref[...] = acc


def kernel(x, Wq, K_ext, V_ext, Wo):
    x2 = x[0].astype(jnp.bfloat16)
    wq = Wq.astype(jnp.bfloat16)
    kt = K_ext[0].transpose(1, 0, 2).astype(jnp.bfloat16)
    vt = V_ext[0].transpose(1, 0, 2).astype(jnp.bfloat16)
    wo = Wo.astype(jnp.bfloat16)

    out = pl.pallas_call(
        _body,
        out_shape=jax.ShapeDtypeStruct((SQ, DM), jnp.float32),
        in_specs=[pl.BlockSpec(memory_space=pltpu.VMEM)] * 5,
        out_specs=pl.BlockSpec(memory_space=pltpu.VMEM),
        scratch_shapes=[
            pltpu.VMEM((N_DEV, HQ, SKV_SHARD, DH), jnp.bfloat16),
            pltpu.VMEM((N_DEV, HQ, SKV_SHARD, DH), jnp.bfloat16),
            pltpu.VMEM((HQ, SQ, DH), jnp.bfloat16),
            pltpu.VMEM((HQ, SQ, DH), jnp.bfloat16),
            pltpu.SemaphoreType.DMA((2,)),
            pltpu.SemaphoreType.DMA((N_DEV - 1,)),
            pltpu.SemaphoreType.DMA((N_DEV - 1,)),
            pltpu.SemaphoreType.DMA((N_DEV - 1,)),
            pltpu.SemaphoreType.DMA((N_DEV - 1,)),
        ],
        compiler_params=pltpu.CompilerParams(collective_id=0),
    )(x2, wq, kt, vt, wo)
    return out[None]


# baseline (device time: 600595 ns/iter reference)
import jax
import jax.numpy as jnp
from jax import lax
from jax.experimental import pallas as pl
from jax.experimental.pallas import tpu as pltpu

N_DEV = 4
SQ = 2048
SKV_SHARD = 2048
HQ = 8
DH = 128
DM = HQ * DH
BQ = 128
BLK = 64
SCALE = 0.08838834764831843
NEG = -1e9


def _body(x_ref, wq_ref, k_ref, v_ref, wo_ref, out_ref,
          kg_ref, vg_ref, q_ref, ksend, krecv, vsend, vrecv):
    me = lax.axis_index("i")
    left = lax.rem(me + N_DEV - 1, N_DEV)
    right = lax.rem(me + 1, N_DEV)

    barrier_sem = pltpu.get_barrier_semaphore()
    pl.semaphore_signal(barrier_sem, inc=1, device_id=(left,),
                        device_id_type=pl.DeviceIdType.MESH)
    pl.semaphore_signal(barrier_sem, inc=1, device_id=(right,),
                        device_id_type=pl.DeviceIdType.MESH)
    pl.semaphore_wait(barrier_sem, 2)

    for h in range(N_DEV - 1):
        k_src = k_ref if h == 0 else kg_ref.at[h - 1]
        v_src = v_ref if h == 0 else vg_ref.at[h - 1]
        k_send = pltpu.make_async_remote_copy(
            src_ref=k_src, dst_ref=kg_ref.at[h],
            send_sem=ksend.at[h], recv_sem=krecv.at[h],
            device_id=(right,), device_id_type=pl.DeviceIdType.MESH)
        v_send = pltpu.make_async_remote_copy(
            src_ref=v_src, dst_ref=vg_ref.at[h],
            send_sem=vsend.at[h], recv_sem=vrecv.at[h],
            device_id=(right,), device_id_type=pl.DeviceIdType.MESH)
        k_send.start()
        v_send.start()

        if h == 0:
            for hh in range(HQ):
                qh = jnp.dot(x_ref[...], wq_ref[:, hh * DH:(hh + 1) * DH],
                             preferred_element_type=jnp.float32)
                q_ref[hh] = qh.astype(jnp.bfloat16)

        k_recv = pltpu.make_async_remote_copy(
            src_ref=kg_ref.at[h], dst_ref=kg_ref.at[h],
            send_sem=ksend.at[h], recv_sem=krecv.at[h],
            device_id=(left,), device_id_type=pl.DeviceIdType.MESH)
        v_recv = pltpu.make_async_remote_copy(
            src_ref=vg_ref.at[h], dst_ref=vg_ref.at[h],
            send_sem=vsend.at[h], recv_sem=vrecv.at[h],
            device_id=(left,), device_id_type=pl.DeviceIdType.MESH)
        k_recv.wait_recv()
        v_recv.wait_recv()
        k_send.wait_send()
        v_send.wait_send()

    origins = [me] + [lax.rem(me - h - 1 + N_DEV, N_DEV)
                      for h in range(N_DEV - 1)]
    n_qb = SQ // BQ

    def qb_step(qb, carry):
        row_blk = (me * SQ + qb * BQ
                   + lax.broadcasted_iota(jnp.int32, (BQ, SKV_SHARD), 0)) // BLK
        col = lax.broadcasted_iota(jnp.int32, (BQ, SKV_SHARD), 1)
        biases = []
        for org in origins:
            col_blk = (org * SKV_SHARD + col) // BLK
            keep = ((row_blk == col_blk) | (col_blk == 0)
                    | (lax.rem(row_blk + col_blk, 3) == 0))
            biases.append(jnp.where(keep, 0.0, NEG).astype(jnp.float32))

        def h_step(h, _):
            q = q_ref[h, pl.ds(qb * BQ, BQ), :]
            num = jnp.zeros((BQ, DH), jnp.float32)
            den = jnp.zeros((BQ, 1), jnp.float32)
            for src in range(N_DEV):
                k = k_ref[h] if src == 0 else kg_ref[src - 1, h]
                v = v_ref[h] if src == 0 else vg_ref[src - 1, h]
                s = lax.dot_general(q, k, (((1,), (1,)), ((), ())),
                                    preferred_element_type=jnp.float32)
                w = jnp.exp(s * SCALE + biases[src])
                den = den + w.sum(axis=1, keepdims=True)
                num = num + jnp.dot(w.astype(jnp.bfloat16), v,
                                    preferred_element_type=jnp.float32)
            ctx = (num / den).astype(jnp.bfloat16)
            out_acc = jnp.dot(ctx, wo_ref[pl.ds(h * DH, DH), :],
                              preferred_element_type=jnp.float32)
            out_ref[pl.ds(qb * BQ, BQ), :] += out_acc.astype(jnp.bfloat16)
            return 0

        out_ref[pl.ds(qb * BQ, BQ), :] = jnp.zeros((BQ, DM), jnp.bfloat16)
        lax.fori_loop(0, HQ, h_step, 0)
        return carry

    lax.fori_loop(0, n_qb, qb_step, 0)


def kernel(x, Wq, K_ext, V_ext, Wo):
    x2 = x[0].astype(jnp.bfloat16)
    wq = Wq.astype(jnp.bfloat16)
    kt = K_ext[0].transpose(1, 0, 2).astype(jnp.bfloat16)
    vt = V_ext[0].transpose(1, 0, 2).astype(jnp.bfloat16)
    wo = Wo.astype(jnp.bfloat16)

    out = pl.pallas_call(
        _body,
        out_shape=jax.ShapeDtypeStruct((SQ, DM), jnp.bfloat16),
        in_specs=[pl.BlockSpec(memory_space=pltpu.VMEM)] * 5,
        out_specs=pl.BlockSpec(memory_space=pltpu.VMEM),
        scratch_shapes=[
            pltpu.VMEM((N_DEV - 1, HQ, SKV_SHARD, DH), jnp.bfloat16),
            pltpu.VMEM((N_DEV - 1, HQ, SKV_SHARD, DH), jnp.bfloat16),
            pltpu.VMEM((HQ, SQ, DH), jnp.bfloat16),
            pltpu.SemaphoreType.DMA((N_DEV - 1,)),
            pltpu.SemaphoreType.DMA((N_DEV - 1,)),
            pltpu.SemaphoreType.DMA((N_DEV - 1,)),
            pltpu.SemaphoreType.DMA((N_DEV - 1,)),
        ],
        compiler_params=pltpu.CompilerParams(
            collective_id=0, vmem_limit_bytes=60 * 1024 * 1024),
    )(x2, wq, kt, vt, wo)
    return out[None].astype(jnp.float32)


# device time: 406512 ns/iter; 1.4774x vs baseline; 1.4774x over previous
import jax
import jax.numpy as jnp
from jax import lax
from jax.experimental import pallas as pl
from jax.experimental.pallas import tpu as pltpu

N_DEV = 4
SQ = 2048
SKV_SHARD = 2048
HQ = 8
DH = 128
DM = HQ * DH
BQ = 128
BLK = 64
SCALE = 0.08838834764831843


def _qproj_body(x_ref, wq_ref, q_ref):
    for h in range(HQ):
        qh = jnp.dot(x_ref[...], wq_ref[:, h * DH:(h + 1) * DH],
                     preferred_element_type=jnp.float32)
        q_ref[h] = qh.astype(jnp.bfloat16)


def _body(q_ref, k_ref, v_ref, wo_ref, out_ref,
          kg_ref, vg_ref, num_ref, den_ref, ksend, krecv, vsend, vrecv):
    me = lax.axis_index("i")
    left = lax.rem(me + N_DEV - 1, N_DEV)
    right = lax.rem(me + 1, N_DEV)

    barrier_sem = pltpu.get_barrier_semaphore()
    pl.semaphore_signal(barrier_sem, inc=1, device_id=(left,),
                        device_id_type=pl.DeviceIdType.MESH)
    pl.semaphore_signal(barrier_sem, inc=1, device_id=(right,),
                        device_id_type=pl.DeviceIdType.MESH)
    pl.semaphore_wait(barrier_sem, 2)

    def make_send(h):
        k_src = k_ref if h == 0 else kg_ref.at[h - 1]
        v_src = v_ref if h == 0 else vg_ref.at[h - 1]
        k_send = pltpu.make_async_remote_copy(
            src_ref=k_src, dst_ref=kg_ref.at[h],
            send_sem=ksend.at[h], recv_sem=krecv.at[h],
            device_id=(right,), device_id_type=pl.DeviceIdType.MESH)
        v_send = pltpu.make_async_remote_copy(
            src_ref=v_src, dst_ref=vg_ref.at[h],
            send_sem=vsend.at[h], recv_sem=vrecv.at[h],
            device_id=(right,), device_id_type=pl.DeviceIdType.MESH)
        return k_send, v_send

    def process(get_k, get_v, org, init):

        def qb_step(qb, carry):
            rb = (me * SQ + qb * BQ
                  + lax.broadcasted_iota(jnp.int32, (BQ, 1), 0)) // BLK
            cb = (org * SKV_SHARD
                  + lax.broadcasted_iota(jnp.int32, (1, SKV_SHARD), 1)) // BLK
            s3 = lax.rem(rb, 3) + lax.rem(cb, 3)
            keep = (rb == cb) | (cb == 0) | (s3 == 0) | (s3 == 3)
            keepf = keep.astype(jnp.bfloat16)
            rows = pl.ds(qb * BQ, BQ)

            def h_step(h, hcarry):
                q = q_ref[h, rows, :]
                s = lax.dot_general(q, get_k(h), (((1,), (1,)), ((), ())),
                                    preferred_element_type=jnp.float32)
                w = jnp.exp(s.astype(jnp.bfloat16)) * keepf
                d = jnp.sum(w, axis=1, keepdims=True, dtype=jnp.float32)
                d = jnp.broadcast_to(d, (BQ, DH))
                n = jnp.dot(w, get_v(h),
                            preferred_element_type=jnp.float32)
                if init:
                    num_ref[h, rows, :] = n
                    den_ref[h, rows, :] = d
                else:
                    num_ref[h, rows, :] += n
                    den_ref[h, rows, :] += d
                return hcarry

            lax.fori_loop(0, HQ, h_step, 0)
            return carry

        lax.fori_loop(0, SQ // BQ, qb_step, 0)

    sends = [make_send(0)]
    sends[0][0].start()
    sends[0][1].start()
    process(lambda h: k_ref[h], lambda h: v_ref[h], me, init=True)

    for h in range(N_DEV - 1):
        k_recv = pltpu.make_async_remote_copy(
            src_ref=kg_ref.at[h], dst_ref=kg_ref.at[h],
            send_sem=ksend.at[h], recv_sem=krecv.at[h],
            device_id=(left,), device_id_type=pl.DeviceIdType.MESH)
        v_recv = pltpu.make_async_remote_copy(
            src_ref=vg_ref.at[h], dst_ref=vg_ref.at[h],
            send_sem=vsend.at[h], recv_sem=vrecv.at[h],
            device_id=(left,), device_id_type=pl.DeviceIdType.MESH)
        k_recv.wait_recv()
        v_recv.wait_recv()
        if h < N_DEV - 2:
            nxt = make_send(h + 1)
            nxt[0].start()
            nxt[1].start()
            sends.append(nxt)
        org = lax.rem(me - h - 1 + N_DEV, N_DEV)
        process(lambda hh: kg_ref[h, hh], lambda hh: vg_ref[h, hh],
                org, init=False)

    for k_send, v_send in sends:
        k_send.wait_send()
        v_send.wait_send()

    RCH = 512
    for r0 in range(0, SQ, RCH):
        acc = jnp.zeros((RCH, DM), jnp.float32)
        for h in range(HQ):
            ctx = (num_ref[h, r0:r0 + RCH, :]
                   / den_ref[h, r0:r0 + RCH, :]).astype(jnp.bfloat16)
            acc = acc + jnp.dot(ctx, wo_ref[h * DH:(h + 1) * DH, :],
                                preferred_element_type=jnp.float32)
        out_ref[r0:r0 + RCH, :] = acc.astype(jnp.bfloat16)


def kernel(x, Wq, K_ext, V_ext, Wo):
    x2 = x[0].astype(jnp.bfloat16)
    wq = (Wq * SCALE).astype(jnp.bfloat16)
    kt = K_ext[0].transpose(1, 0, 2).astype(jnp.bfloat16)
    vt = V_ext[0].transpose(1, 0, 2).astype(jnp.bfloat16)
    wo = Wo.astype(jnp.bfloat16)

    q = pl.pallas_call(
        _qproj_body,
        out_shape=jax.ShapeDtypeStruct((HQ, SQ, DH), jnp.bfloat16),
        in_specs=[pl.BlockSpec(memory_space=pltpu.VMEM)] * 2,
        out_specs=pl.BlockSpec(memory_space=pltpu.VMEM),
        compiler_params=pltpu.CompilerParams(
            vmem_limit_bytes=32 * 1024 * 1024),
    )(x2, wq)

    out = pl.pallas_call(
        _body,
        out_shape=jax.ShapeDtypeStruct((SQ, DM), jnp.bfloat16),
        in_specs=[pl.BlockSpec(memory_space=pltpu.VMEM)] * 4,
        out_specs=pl.BlockSpec(memory_space=pltpu.VMEM),
        scratch_shapes=[
            pltpu.VMEM((N_DEV - 1, HQ, SKV_SHARD, DH), jnp.bfloat16),
            pltpu.VMEM((N_DEV - 1, HQ, SKV_SHARD, DH), jnp.bfloat16),
            pltpu.VMEM((HQ, SQ, DH), jnp.float32),
            pltpu.VMEM((HQ, SQ, DH), jnp.float32),
            pltpu.SemaphoreType.DMA((N_DEV - 1,)),
            pltpu.SemaphoreType.DMA((N_DEV - 1,)),
            pltpu.SemaphoreType.DMA((N_DEV - 1,)),
            pltpu.SemaphoreType.DMA((N_DEV - 1,)),
        ],
        compiler_params=pltpu.CompilerParams(
            collective_id=0, vmem_limit_bytes=63 * 1024 * 1024),
    )(q, kt, vt, wo)
    return out[None].astype(jnp.float32)
